# Initial kernel scaffold; baseline (speedup 1.0000x reference)
import jax
import jax.numpy as jnp
from jax.experimental import pallas as pl


def kernel(means, precisions, targets):
    raise NotImplementedError("write your pallas kernel here")


# speedup vs baseline: 3.3322x; 3.3322x over previous
import functools
import math

import jax
import jax.numpy as jnp
from jax import lax
from jax.experimental import pallas as pl
from jax.experimental.pallas import tpu as pltpu
from jax.experimental.pallas import tpu_sc as plsc

B = 8
N = 2048
D = 512
C = 64
DC = 4
DW = D // DC
ECH = 128
NCH = N // ECH
L = 16

LN2 = math.log(2.0)
LOG2PI = math.log(2.0 * math.pi)


_GATHER_DNUMS = lax.GatherDimensionNumbers(
    offset_dims=(), collapsed_slice_dims=(0,), start_index_map=(0,))


def _lane_shuffle(v, idx):
    return lax.gather(v, idx[:, None], _GATHER_DNUMS, (1,),
                      mode=lax.GatherScatterMode.PROMISE_IN_BOUNDS)


def _hsum(v):
    iota = lax.iota(jnp.int32, L)
    for sh in (8, 4, 2, 1):
        v = v + _lane_shuffle(v, iota ^ sh)
    return v


def _vlog(x):
    bits = lax.bitcast_convert_type(x, jnp.int32)
    e = ((bits >> 23) & 0xFF) - 127
    mbits = (bits & 0x7FFFFF) | (127 << 23)
    m = lax.bitcast_convert_type(mbits, jnp.float32)
    s = (m - 1.0) / (m + 1.0)
    z = s * s
    p = 2.0 / 5.0 + z * (2.0 / 7.0)
    p = 2.0 / 3.0 + z * p
    p = 2.0 + z * p
    return e.astype(jnp.float32) * LN2 + s * p


def _make_sc_call():
    mesh = plsc.VectorSubcoreMesh(core_axis_name="c", subcore_axis_name="s")

    @functools.partial(
        pl.kernel,
        out_type=[
            jax.ShapeDtypeStruct((B, C, D), jnp.float32),
            jax.ShapeDtypeStruct((B, C, D), jnp.float32),
            jax.ShapeDtypeStruct((B, DC, C, L), jnp.float32),
        ],
        mesh=mesh,
        scratch_types=[
            pltpu.VMEM((ECH, DW), jnp.float32),
            pltpu.VMEM((ECH, DW), jnp.float32),
            pltpu.VMEM((ECH,), jnp.int32),
            pltpu.VMEM((C, DW), jnp.float32),
            pltpu.VMEM((C, DW), jnp.float32),
            pltpu.VMEM((C, L), jnp.float32),
            pltpu.VMEM((C, L), jnp.float32),
            pltpu.VMEM((C, L), jnp.float32),
            pltpu.VMEM((C, L), jnp.float32),
        ],
    )
    def sc_kernel(means_h, prec_h, tgt_h, outP_h, outM_h, part_h,
                  m_buf, p_buf, t_buf, accP, accPM, slog, sq, cnt, contrib):
        cid = lax.axis_index("c")
        sid = lax.axis_index("s")
        wid = cid * 16 + sid
        b = wid // DC
        dc = wid % DC
        d0 = dc * DW

        zero = jnp.zeros((L,), jnp.float32)
        ones = jnp.ones((L,), jnp.float32)

        def zero_body(c, _):
            for j in range(DW // L):
                sl = pl.ds(j * L, L)
                accP[c, sl] = zero
                accPM[c, sl] = zero
            slog[c, :] = zero
            sq[c, :] = zero
            cnt[c, :] = zero
            return 0

        lax.fori_loop(0, C, zero_body, 0)

        def chunk_body(ch, _):
            e0 = ch * ECH
            pltpu.sync_copy(means_h.at[b, pl.ds(e0, ECH), pl.ds(d0, DW)], m_buf)
            pltpu.sync_copy(prec_h.at[b, pl.ds(e0, ECH), pl.ds(d0, DW)], p_buf)
            pltpu.sync_copy(tgt_h.at[b, pl.ds(e0, ECH)], t_buf)

            def ex_group(g, _):
                tvec = t_buf[pl.ds(g * L, L)]
                for e16 in range(L):
                    e = g * L + e16
                    t = tvec[e16]
                    slog_part = zero
                    sq_part = zero
                    for j in range(DW // L):
                        sl = pl.ds(j * L, L)
                        p = p_buf[e, sl]
                        m = m_buf[e, sl]
                        pm = p * m
                        plsc.addupdate(accP.at[t, sl], p)
                        plsc.addupdate(accPM.at[t, sl], pm)
                        slog_part = slog_part + _vlog(p)
                        sq_part = sq_part + pm * m
                    plsc.addupdate(slog.at[t], slog_part)
                    plsc.addupdate(sq.at[t], sq_part)
                    plsc.addupdate(cnt.at[t], ones)
                return 0

            lax.fori_loop(0, ECH // L, ex_group, 0)
            return 0

        lax.fori_loop(0, NCH, chunk_body, 0)

        def fin_body(c, _):
            logP_part = zero
            pmq_part = zero
            for j in range(DW // L):
                sl = pl.ds(j * L, L)
                P = accP[c, sl]
                PM = accPM[c, sl]
                mean = PM / P
                accPM[c, sl] = mean
                logP_part = logP_part + _vlog(P)
                pmq_part = pmq_part + PM * mean
            n = jnp.maximum(_hsum(cnt[c, :]) * (1.0 / L), 1.0)
            val = (0.5 * (1.0 - n) * (DW * LOG2PI)
                   + 0.5 * (_hsum(slog[c, :]) - _hsum(logP_part))
                   + 0.5 * (_hsum(pmq_part) - _hsum(sq[c, :])))
            contrib[c, :] = val
            return 0

        lax.fori_loop(0, C, fin_body, 0)

        pltpu.sync_copy(accP, outP_h.at[b, :, pl.ds(d0, DW)])
        pltpu.sync_copy(accPM, outM_h.at[b, :, pl.ds(d0, DW)])
        pltpu.sync_copy(contrib, part_h.at[b, dc])

    return sc_kernel


_sc_call = _make_sc_call()


def kernel(means, precisions, targets):
    outP, outM, part = _sc_call(means, precisions, targets.astype(jnp.int32))
    log_norm = part[:, :, :, 0].sum(axis=1)
    return (outM, outP, log_norm)


# speedup vs baseline: 4.9327x; 1.4803x over previous
import functools
import math

import jax
import jax.numpy as jnp
from jax import lax
from jax.experimental import pallas as pl
from jax.experimental.pallas import tpu as pltpu
from jax.experimental.pallas import tpu_sc as plsc

B = 8
N = 2048
D = 512
C = 64
DC = 4
DW = D // DC
ECH = 128
NCH = N // ECH
L = 16

LN2 = math.log(2.0)
LOG2PI = math.log(2.0 * math.pi)

_C5 = 0.030449004538668337
_C4 = -0.13158182508875452
_C3 = 0.28527268109056625
_C2 = -0.4902307234234066
_C1 = 0.3060883032733293
_C0 = 9.97503255216024e-06 - 127.0 * LN2
_K = LN2 / (2.0 ** 23)

_GATHER_DNUMS = lax.GatherDimensionNumbers(
    offset_dims=(), collapsed_slice_dims=(0,), start_index_map=(0,))


def _lane_shuffle(v, idx):
    return lax.gather(v, idx[:, None], _GATHER_DNUMS, (1,),
                      mode=lax.GatherScatterMode.PROMISE_IN_BOUNDS)


def _hsum(v):
    iota = lax.iota(jnp.int32, L)
    for sh in (8, 4, 2, 1):
        v = v + _lane_shuffle(v, iota ^ sh)
    return v


def _vlog(x):
    bits = lax.bitcast_convert_type(x, jnp.int32)
    bf = bits.astype(jnp.float32)
    t = (bits & 0x7FFFFF).astype(jnp.float32) * (2.0 ** -23)
    h = _C5 * t + _C4
    h = h * t + _C3
    h = h * t + _C2
    h = h * t + _C1
    h = h * t + _C0
    return bf * _K + h


def _make_sc_call():
    mesh = plsc.VectorSubcoreMesh(core_axis_name="c", subcore_axis_name="s")

    @functools.partial(
        pl.kernel,
        out_type=[
            jax.ShapeDtypeStruct((B, C, D), jnp.float32),
            jax.ShapeDtypeStruct((B, C, D), jnp.float32),
            jax.ShapeDtypeStruct((B, DC, C, L), jnp.float32),
        ],
        mesh=mesh,
        scratch_types=[
            pltpu.VMEM((2, ECH, DW), jnp.float32),
            pltpu.VMEM((2, ECH, DW), jnp.float32),
            pltpu.VMEM((2, ECH), jnp.int32),
            pltpu.VMEM((C, DW), jnp.float32),
            pltpu.VMEM((C, DW), jnp.float32),
            pltpu.VMEM((C, L), jnp.float32),
            pltpu.VMEM((C, L), jnp.float32),
            pltpu.VMEM((C, L), jnp.float32),
            pltpu.VMEM((C, L), jnp.float32),
            pltpu.SemaphoreType.DMA((2,)),
        ],
    )
    def sc_kernel(means_h, prec_h, tgt_h, outP_h, outM_h, part_h,
                  m_buf, p_buf, t_buf, accP, accPM, slog, sq, cnt, contrib,
                  sem):
        cid = lax.axis_index("c")
        sid = lax.axis_index("s")
        wid = cid * 16 + sid
        b = wid // DC
        dc = wid % DC
        d0 = dc * DW

        zero = jnp.zeros((L,), jnp.float32)
        ones = jnp.ones((L,), jnp.float32)

        def copies(ch, slot):
            e0 = ch * ECH
            return (
                pltpu.make_async_copy(
                    means_h.at[b, pl.ds(e0, ECH), pl.ds(d0, DW)],
                    m_buf.at[slot], sem.at[slot]),
                pltpu.make_async_copy(
                    prec_h.at[b, pl.ds(e0, ECH), pl.ds(d0, DW)],
                    p_buf.at[slot], sem.at[slot]),
                pltpu.make_async_copy(
                    tgt_h.at[b, pl.ds(e0, ECH)],
                    t_buf.at[slot], sem.at[slot]),
            )

        def issue(ch, slot):
            for c_ in copies(ch, slot):
                c_.start()

        def wait(ch, slot):
            for c_ in copies(ch, slot):
                c_.wait()

        def zero_body(c, _):
            for j in range(DW // L):
                sl = pl.ds(j * L, L)
                accP[c, sl] = zero
                accPM[c, sl] = zero
            slog[c, :] = zero
            sq[c, :] = zero
            cnt[c, :] = zero
            return 0

        issue(0, 0)
        lax.fori_loop(0, C, zero_body, 0)

        def chunk_body(ch, _):
            slot = lax.rem(ch, 2)

            @pl.when(ch + 1 < NCH)
            def _():
                issue(ch + 1, 1 - slot)

            wait(ch, slot)

            def ex_group(g, _):
                tvec = t_buf[slot, pl.ds(g * L, L)]
                for e16 in range(L):
                    e = g * L + e16
                    t = tvec[e16]
                    slog_part = zero
                    sq_part = zero
                    for j in range(DW // L):
                        sl = pl.ds(j * L, L)
                        p = p_buf[slot, e, sl]
                        m = m_buf[slot, e, sl]
                        pm = p * m
                        plsc.addupdate(accP.at[t, sl], p)
                        plsc.addupdate(accPM.at[t, sl], pm)
                        slog_part = slog_part + _vlog(p)
                        sq_part = sq_part + pm * m
                    plsc.addupdate(slog.at[t], slog_part)
                    plsc.addupdate(sq.at[t], sq_part)
                    plsc.addupdate(cnt.at[t], ones)
                return 0

            lax.fori_loop(0, ECH // L, ex_group, 0)
            return 0

        lax.fori_loop(0, NCH, chunk_body, 0)

        def fin_body(c, _):
            logP_part = zero
            pmq_part = zero
            for j in range(DW // L):
                sl = pl.ds(j * L, L)
                P = accP[c, sl]
                PM = accPM[c, sl]
                mean = PM / P
                accPM[c, sl] = mean
                logP_part = logP_part + _vlog(P)
                pmq_part = pmq_part + PM * mean
            n = jnp.maximum(_hsum(cnt[c, :]) * (1.0 / L), 1.0)
            val = (0.5 * (1.0 - n) * (DW * LOG2PI)
                   + 0.5 * (_hsum(slog[c, :]) - _hsum(logP_part))
                   + 0.5 * (_hsum(pmq_part) - _hsum(sq[c, :])))
            contrib[c, :] = val
            return 0

        lax.fori_loop(0, C, fin_body, 0)

        pltpu.sync_copy(accP, outP_h.at[b, :, pl.ds(d0, DW)])
        pltpu.sync_copy(accPM, outM_h.at[b, :, pl.ds(d0, DW)])
        pltpu.sync_copy(contrib, part_h.at[b, dc])

    return sc_kernel


_sc_call = _make_sc_call()


def kernel(means, precisions, targets):
    outP, outM, part = _sc_call(means, precisions, targets.astype(jnp.int32))
    log_norm = part[:, :, :, 0].sum(axis=1)
    return (outM, outP, log_norm)
